# baseline (speedup 1.0000x reference)
import functools

import jax
import jax.numpy as jnp
from jax import lax
from jax.experimental import pallas as pl
from jax.experimental.pallas import tpu as pltpu
from jax.experimental.pallas import tpu_sc as plsc

NUM_ANGLES = 100000
EMBED_DIM = 32
PAD_DIM = 128
BATCH = 16384
HIST = 200

B = BATCH * HIST
NC = 2
NS = 16
NW = NC * NS
IDX_MINOR = 160
ROWS_PER_GROUP = 1600
K = ROWS_PER_GROUP // IDX_MINOR
GROUPS = B // (NW * ROWS_PER_GROUP)
IDX_ROWS_PER_WORKER = B // (NW * IDX_MINOR)


def _make_sc_gather():
    mesh = plsc.VectorSubcoreMesh(core_axis_name="c", subcore_axis_name="s")

    @functools.partial(
        pl.kernel,
        mesh=mesh,
        out_type=jax.ShapeDtypeStruct((B, PAD_DIM), jnp.float32),
        scratch_types=[
            pltpu.VMEM((K, IDX_MINOR), jnp.int32),
            pltpu.VMEM((K, IDX_MINOR), jnp.int32),
            pltpu.VMEM((ROWS_PER_GROUP, EMBED_DIM), jnp.float32),
            pltpu.VMEM((ROWS_PER_GROUP, EMBED_DIM), jnp.float32),
            pltpu.SemaphoreType.DMA,
            pltpu.SemaphoreType.DMA,
            pltpu.SemaphoreType.DMA,
            pltpu.SemaphoreType.DMA,
        ],
        compiler_params=pltpu.CompilerParams(use_tc_tiling_on_sc=False),
    )
    def gather_kernel(idx_hbm, table_hbm, out_hbm,
                      idx0, idx1, rows0, rows1,
                      gsem0, gsem1, ssem0, ssem1):
        wid = lax.axis_index("s") * NC + lax.axis_index("c")
        idx_row_base = wid * IDX_ROWS_PER_WORKER
        out_base = wid * IDX_ROWS_PER_WORKER * IDX_MINOR

        def out_slice(g):
            return out_hbm.at[pl.ds(out_base + g * ROWS_PER_GROUP,
                                    ROWS_PER_GROUP), pl.ds(0, EMBED_DIM)]

        def load_and_fire(g, idx_v, rows_v, gsem):
            pltpu.sync_copy(idx_hbm.at[pl.ds(idx_row_base + g * K, K)], idx_v)
            return [
                pltpu.async_copy(
                    table_hbm.at[idx_v.at[j]],
                    rows_v.at[pl.ds(j * IDX_MINOR, IDX_MINOR)],
                    gsem,
                )
                for j in range(K)
            ]

        h0 = load_and_fire(0, idx0, rows0, gsem0)
        h1 = load_and_fire(1, idx1, rows1, gsem1)
        for h in h0:
            h.wait()
        pltpu.async_copy(rows0, out_slice(0), ssem0)
        for h in h1:
            h.wait()
        pltpu.async_copy(rows1, out_slice(1), ssem1)

        def body(gg, _):
            a = 2 * gg
            b = a + 1
            pltpu.make_async_copy(rows0, out_slice(a), ssem0).wait()
            ha = load_and_fire(a, idx0, rows0, gsem0)
            pltpu.make_async_copy(rows1, out_slice(b), ssem1).wait()
            hb = load_and_fire(b, idx1, rows1, gsem1)
            for h in ha:
                h.wait()
            pltpu.async_copy(rows0, out_slice(a), ssem0)
            for h in hb:
                h.wait()
            pltpu.async_copy(rows1, out_slice(b), ssem1)
            return ()

        lax.fori_loop(1, GROUPS // 2, body, (), unroll=False)

        pltpu.make_async_copy(rows0, out_slice(0), ssem0).wait()
        pltpu.make_async_copy(rows1, out_slice(1), ssem1).wait()

    return gather_kernel


_sc_gather = _make_sc_gather()


def kernel(theta, table):
    idx2d = theta.reshape(B // IDX_MINOR, IDX_MINOR).astype(jnp.int32)
    out_pad = _sc_gather(idx2d, table)
    return out_pad.reshape(BATCH, HIST, PAD_DIM)[:, :, :EMBED_DIM]
